# Initial kernel scaffold; baseline (speedup 1.0000x reference)
import jax
import jax.numpy as jnp
from jax.experimental import pallas as pl


def kernel(inp, history, table, nodes_min, nodes_extent):
    raise NotImplementedError("write your pallas kernel here")


# speedup vs baseline: 17.0129x; 17.0129x over previous
import functools

import jax
import jax.numpy as jnp
from jax import lax
from jax.experimental import pallas as pl
from jax.experimental.pallas import tpu as pltpu
from jax.experimental.pallas import tpu_sc as plsc

R = 131072
DEPTH = 8
DIM = 8
NPTS = 4
TABLE_SIZE = 4194304

_PIS = (774363409, 2654435761, 805459861, 100000007,
        334363391, 1334363413, 734363407, 2134363393)
K_XOR = tuple(((c + 1) * _PIS[c]) % TABLE_SIZE for c in range(8))

NC = 2
NS = 16
NW = NC * NS
RPW = R // NW
B_R = 128
NBLK = RPW // B_R
NG = B_R // 16


def _splat(v):
    return jnp.broadcast_to(jnp.asarray(v, jnp.int32), (16,))


def _sc_body(table_h, hist_h, inp_h, nodes_h, out_h,
             jall_v, j_v, tidx_v, node_v, feat_v, inp_v, out_v, sem):
    wid = lax.axis_index("s") * NC + lax.axis_index("c")
    iota = lax.iota(jnp.int32, 16)

    @pl.loop(jnp.int32(0), jnp.int32(NBLK))
    def block_body(b):
        base = wid * RPW + b * B_R
        for row in range(DEPTH):
            pltpu.sync_copy(hist_h.at[jnp.int32(row), pl.ds(base, B_R)],
                            jall_v.at[jnp.int32(row)])
        for row in range(3 * NPTS):
            pltpu.sync_copy(inp_h.at[jnp.int32(row), pl.ds(base, B_R)],
                            inp_v.at[jnp.int32(row)])

        @pl.loop(jnp.int32(0), jnp.int32(DEPTH))
        def depth_body(i):
            @pl.loop(jnp.int32(0), jnp.int32(NG))
            def tidx_body(g):
                lane = g * 16 + iota
                jv = plsc.load_gather(jall_v, [_splat(i), lane])
                j_v[pl.ds(g * 16, 16)] = jv
                for c in range(8):
                    tidx_v[jnp.int32(c), pl.ds(g * 16, 16)] = jv ^ jnp.int32(K_XOR[c])

            descs = [pltpu.async_copy(nodes_h.at[j_v], node_v, sem)]
            for c in range(8):
                descs.append(
                    pltpu.async_copy(table_h.at[tidx_v.at[jnp.int32(c)]],
                                     feat_v.at[jnp.int32(c)], sem))
            for dsc in descs:
                dsc.wait()

            @pl.loop(jnp.int32(0), jnp.int32(NG))
            def group_body(g):
                lane = g * 16 + iota
                nm = [plsc.load_gather(node_v, [lane, _splat(comp)])
                      for comp in range(3)]
                ie = [1.0 / plsc.load_gather(node_v, [lane, _splat(3 + comp)])
                      for comp in range(3)]
                w = []
                for k in range(NPTS):
                    px = jnp.clip((inp_v[jnp.int32(3 * k + 0), pl.ds(g * 16, 16)] - nm[0]) * ie[0], 0.0, 1.0)
                    py = jnp.clip((inp_v[jnp.int32(3 * k + 1), pl.ds(g * 16, 16)] - nm[1]) * ie[1], 0.0, 1.0)
                    pz = jnp.clip((inp_v[jnp.int32(3 * k + 2), pl.ds(g * 16, 16)] - nm[2]) * ie[2], 0.0, 1.0)
                    ax, ay, az = 1.0 - px, 1.0 - py, 1.0 - pz
                    b00, b10, b01, b11 = ax * ay, px * ay, ax * py, px * py
                    w.append((b00 * az, b10 * az, b01 * az, b00 * pz,
                              b10 * pz, b01 * pz, b11 * az, b11 * pz))
                col0 = i * 32
                for d in range(DIM):
                    f = [plsc.load_gather(feat_v, [_splat(c), lane, _splat(d)])
                         for c in range(8)]
                    for k in range(NPTS):
                        acc = w[k][0] * f[0]
                        for c in range(1, 8):
                            acc = acc + w[k][c] * f[c]
                        col = jnp.broadcast_to(col0 + (k * 8 + d), (16,))
                        plsc.store_scatter(out_v, [lane, col], acc)

        pltpu.sync_copy(out_v, out_h.at[pl.ds(base, B_R)])


@functools.cache
def _build_sc_kernel():
    return pl.kernel(
        _sc_body,
        out_type=jax.ShapeDtypeStruct((R, DEPTH * NPTS * DIM), jnp.float32),
        mesh=plsc.VectorSubcoreMesh(core_axis_name="c", subcore_axis_name="s",
                                    num_cores=NC, num_subcores=NS),
        compiler_params=pltpu.CompilerParams(use_tc_tiling_on_sc=False,
                                             needs_layout_passes=False),
        scratch_types=[
            pltpu.VMEM((DEPTH, B_R), jnp.int32),
            pltpu.VMEM((B_R,), jnp.int32),
            pltpu.VMEM((8, B_R), jnp.int32),
            pltpu.VMEM((B_R, 8), jnp.float32),
            pltpu.VMEM((8, B_R, DIM), jnp.float32),
            pltpu.VMEM((3 * NPTS, B_R), jnp.float32),
            pltpu.VMEM((B_R, DEPTH * NPTS * DIM), jnp.float32),
            pltpu.SemaphoreType.DMA,
        ],
    )


def kernel(inp, history, table, nodes_min, nodes_extent):
    inp_t = inp.astype(jnp.float32).reshape(R, 3 * NPTS).T
    hist_t = history.astype(jnp.int32).T
    nodes_cat = jnp.concatenate(
        [nodes_min.astype(jnp.float32), nodes_extent.astype(jnp.float32),
         jnp.zeros((nodes_min.shape[0], 2), jnp.float32)], axis=1)
    return _build_sc_kernel()(table.astype(jnp.float32), hist_t, inp_t, nodes_cat)
